# baseline (speedup 1.0000x reference)
import functools

import jax
import jax.numpy as jnp
from jax import lax
from jax.experimental import pallas as pl
from jax.experimental.pallas import tpu as pltpu
from jax.experimental.pallas import tpu_sc as plsc

DIM = 32
NC, NS = 2, 16
NW = NC * NS
SUBB = 4
TBLK = 4 * SUBB * 512
PASS = 128


def _rsqrt16(x):
    i = plsc.bitcast(x, jnp.int32)
    i = jnp.int32(0x5F3759DF) - (i >> 1)
    y = plsc.bitcast(i, jnp.float32)
    for _ in range(3):
        y = y * (1.5 - 0.5 * x * y * y)
    return y


def _repack_body(u_ref, i_ref, ou_ref, oi_ref):
    for ref, out in ((u_ref, ou_ref), (i_ref, oi_ref)):
        for i in range(TBLK // 512):
            y = jnp.concatenate(
                [ref[:, pl.ds((4 * i + s) * 128, 128)] for s in range(4)],
                axis=0)
            out[pl.ds(i * 128, 128), :] = y.T.astype(jnp.bfloat16)


@functools.cache
def _repack_fn(V):
    grid = (V + TBLK - 1) // TBLK
    return pl.pallas_call(
        _repack_body,
        grid=(grid,),
        in_specs=[pl.BlockSpec((DIM, TBLK), lambda i: (0, i))] * 2,
        out_specs=[pl.BlockSpec((SUBB * 512, 128), lambda i: (i, 0))] * 2,
        out_shape=[jax.ShapeDtypeStruct(
            (grid * SUBB * 512, 128), jnp.bfloat16)] * 2,
    )


@functools.cache
def _fused_fn(B):
    b_w = B // NW
    n_pass = b_w // PASS
    mesh = plsc.VectorSubcoreMesh(
        core_axis_name="c", subcore_axis_name="s",
        num_cores=NC, num_subcores=NS)

    def body(idx_hbm, utab_hbm, itab_hbm, out_hbm,
             idx_v, poff_v, widx, rows_u, rows_p, rows_n, out_v, sem):
        wid = lax.axis_index("s") * NC + lax.axis_index("c")
        pltpu.sync_copy(idx_hbm.at[wid], idx_v)

        tabs = ((utab_hbm, rows_u, 0), (itab_hbm, rows_p, 1),
                (itab_hbm, rows_n, 2))

        def gen_and_fire(p, buf):
            def gen(s, x):
                for _, _, t in tabs:
                    ids = idx_v[pl.ds(t * b_w + p * PASS + s * 16, 16)]
                    row = ((ids >> 9) << 7) + (ids & 127)
                    poff = ((ids >> 7) & 3) << 5
                    widx[buf, t, pl.ds(s * 16, 16)] = row
                    poff_v[buf, t, pl.ds(s * 16, 16)] = poff
                return x
            lax.fori_loop(0, PASS // 16, gen, 0)
            for tab, rows, t in tabs:
                pltpu.async_copy(tab.at[widx.at[buf, t]], rows.at[buf], sem)

        gen_and_fire(0, 0)

        eps = jnp.float32(1e-12)
        lane = lax.iota(jnp.int32, 16)

        def step(p, x):
            buf = lax.rem(p, 2)
            nbuf = lax.rem(p + 1, 2)

            @pl.when(p + 1 < n_pass)
            def _():
                gen_and_fire(p + 1, nbuf)

            for tab, rows, t in tabs:
                pltpu.make_async_copy(
                    tab.at[widx.at[buf, t]], rows.at[buf], sem).wait()

            def sub(s, y):
                pu = poff_v[buf, 0, pl.ds(s * 16, 16)]
                pp_ = poff_v[buf, 1, pl.ds(s * 16, 16)]
                pn = poff_v[buf, 2, pl.ds(s * 16, 16)]
                zero = jnp.zeros((16,), jnp.float32)
                uu = pp = nn = up = un = zero
                for r in range(16):
                    il = s * 16 + r
                    u0, u1 = plsc.unpack(
                        rows_u[buf, il, pl.ds(pu[r], 32)],
                        format=plsc.PackFormat.INTERLEAVED)
                    p0, p1 = plsc.unpack(
                        rows_p[buf, il, pl.ds(pp_[r], 32)],
                        format=plsc.PackFormat.INTERLEAVED)
                    n0, n1 = plsc.unpack(
                        rows_n[buf, il, pl.ds(pn[r], 32)],
                        format=plsc.PackFormat.INTERLEAVED)
                    m = lane == r
                    uu = jnp.where(m, jnp.sum(u0 * u0 + u1 * u1), uu)
                    pp = jnp.where(m, jnp.sum(p0 * p0 + p1 * p1), pp)
                    nn = jnp.where(m, jnp.sum(n0 * n0 + n1 * n1), nn)
                    up = jnp.where(m, jnp.sum(u0 * p0 + u1 * p1), up)
                    un = jnp.where(m, jnp.sum(u0 * n0 + u1 * n1), un)
                uu = jnp.maximum(uu, eps)
                pos = up * _rsqrt16(uu * jnp.maximum(pp, eps))
                neg = un * _rsqrt16(uu * jnp.maximum(nn, eps))
                out_v[pl.ds(p * PASS + s * 16, 16)] = (
                    jnp.maximum(neg - pos + 1.0, 0.0))
                return y

            lax.fori_loop(0, PASS // 16, sub, 0)
            return x

        lax.fori_loop(0, n_pass, step, 0)
        pltpu.sync_copy(out_v, out_hbm.at[pl.ds(wid * b_w, b_w)])

    return pl.kernel(
        body,
        out_type=jax.ShapeDtypeStruct((B,), jnp.float32),
        mesh=mesh,
        compiler_params=pltpu.CompilerParams(
            use_tc_tiling_on_sc=False, needs_layout_passes=False),
        scratch_types=[
            pltpu.VMEM((3 * b_w,), jnp.int32),
            pltpu.VMEM((2, 3, PASS), jnp.int32),
            pltpu.VMEM((2, 3, PASS), jnp.int32),
            pltpu.VMEM((2, PASS, 128), jnp.bfloat16),
            pltpu.VMEM((2, PASS, 128), jnp.bfloat16),
            pltpu.VMEM((2, PASS, 128), jnp.bfloat16),
            pltpu.VMEM((b_w,), jnp.float32),
            pltpu.SemaphoreType.DMA,
        ],
    )


def kernel(user_input, pos_item_input, neg_item_input, user_table, item_table):
    B = user_input.shape[0]
    V = user_table.shape[0]
    b_w = B // NW
    idx = jnp.concatenate(
        [user_input.astype(jnp.int32).reshape(NW, b_w),
         pos_item_input.astype(jnp.int32).reshape(NW, b_w),
         neg_item_input.astype(jnp.int32).reshape(NW, b_w)], axis=1)
    tab_u, tab_i = _repack_fn(V)(user_table.T, item_table.T)
    out = _fused_fn(B)(idx, tab_u, tab_i)
    return out.reshape(B, 1)


# speedup vs baseline: 2.9319x; 2.9319x over previous
import functools

import jax
import jax.numpy as jnp
from jax import lax
from jax.experimental import pallas as pl
from jax.experimental.pallas import tpu as pltpu
from jax.experimental.pallas import tpu_sc as plsc

DIM = 32
NC, NS = 2, 16
NW = NC * NS
SUBB = 8
TBLK = 4 * SUBB * 512
PASS = 128


def _rsqrt16(x):
    i = plsc.bitcast(x, jnp.int32)
    i = jnp.int32(0x5F3759DF) - (i >> 1)
    y = plsc.bitcast(i, jnp.float32)
    for _ in range(3):
        y = y * (1.5 - 0.5 * x * y * y)
    return y


def _repack_body(u_ref, i_ref, ou_ref, oi_ref):
    for ref, out in ((u_ref, ou_ref), (i_ref, oi_ref)):
        for i in range(TBLK // 512):
            y = jnp.concatenate(
                [ref[:, pl.ds((4 * i + s) * 128, 128)] for s in range(4)],
                axis=0)
            out[pl.ds(i * 128, 128), :] = y.T


@functools.cache
def _repack_fn(V):
    grid = (V + TBLK - 1) // TBLK
    return pl.pallas_call(
        _repack_body,
        grid=(grid,),
        in_specs=[pl.BlockSpec((DIM, TBLK), lambda i: (0, i))] * 2,
        out_specs=[pl.BlockSpec((SUBB * 512, 128), lambda i: (i, 0))] * 2,
        out_shape=[jax.ShapeDtypeStruct(
            (grid * SUBB * 512, 128), jnp.float32)] * 2,
    )


@functools.cache
def _fused_fn(B):
    b_w = B // NW
    n_pass = b_w // PASS
    mesh = plsc.VectorSubcoreMesh(
        core_axis_name="c", subcore_axis_name="s",
        num_cores=NC, num_subcores=NS)

    def body(idx_hbm, utab_hbm, itab_hbm, out_hbm,
             idx_v, poff_v, widx, rows_u, rows_p, rows_n, out_v, sem):
        wid = lax.axis_index("s") * NC + lax.axis_index("c")
        pltpu.sync_copy(idx_hbm.at[wid], idx_v)

        tabs = ((utab_hbm, rows_u, 0), (itab_hbm, rows_p, 1),
                (itab_hbm, rows_n, 2))

        def gen_and_fire(p, buf):
            def gen(s, x):
                for _, _, t in tabs:
                    ids = idx_v[pl.ds(t * b_w + p * PASS + s * 16, 16)]
                    row = ((ids >> 9) << 7) + (ids & 127)
                    poff = ((ids >> 7) & 3) << 5
                    widx[buf, t, pl.ds(s * 16, 16)] = row
                    poff_v[buf, t, pl.ds(s * 16, 16)] = poff
                return x
            lax.fori_loop(0, PASS // 16, gen, 0)
            for tab, rows, t in tabs:
                pltpu.async_copy(tab.at[widx.at[buf, t]], rows.at[buf], sem)

        gen_and_fire(0, 0)

        eps = jnp.float32(1e-12)
        lane = lax.iota(jnp.int32, 16)

        def step(p, x):
            buf = lax.rem(p, 2)
            nbuf = lax.rem(p + 1, 2)

            @pl.when(p + 1 < n_pass)
            def _():
                gen_and_fire(p + 1, nbuf)

            for tab, rows, t in tabs:
                pltpu.make_async_copy(
                    tab.at[widx.at[buf, t]], rows.at[buf], sem).wait()

            def sub(s, y):
                pu = poff_v[buf, 0, pl.ds(s * 16, 16)]
                pp_ = poff_v[buf, 1, pl.ds(s * 16, 16)]
                pn = poff_v[buf, 2, pl.ds(s * 16, 16)]
                zero = jnp.zeros((16,), jnp.float32)
                uu = pp = nn = up = un = zero
                for r in range(16):
                    il = s * 16 + r
                    u0 = rows_u[buf, il, pl.ds(pu[r], 16)]
                    u1 = rows_u[buf, il, pl.ds(pu[r] + 16, 16)]
                    p0 = rows_p[buf, il, pl.ds(pp_[r], 16)]
                    p1 = rows_p[buf, il, pl.ds(pp_[r] + 16, 16)]
                    n0 = rows_n[buf, il, pl.ds(pn[r], 16)]
                    n1 = rows_n[buf, il, pl.ds(pn[r] + 16, 16)]
                    m = lane == r
                    uu = jnp.where(m, jnp.sum(u0 * u0 + u1 * u1), uu)
                    pp = jnp.where(m, jnp.sum(p0 * p0 + p1 * p1), pp)
                    nn = jnp.where(m, jnp.sum(n0 * n0 + n1 * n1), nn)
                    up = jnp.where(m, jnp.sum(u0 * p0 + u1 * p1), up)
                    un = jnp.where(m, jnp.sum(u0 * n0 + u1 * n1), un)
                uu = jnp.maximum(uu, eps)
                pos = up * _rsqrt16(uu * jnp.maximum(pp, eps))
                neg = un * _rsqrt16(uu * jnp.maximum(nn, eps))
                out_v[pl.ds(p * PASS + s * 16, 16)] = (
                    jnp.maximum(neg - pos + 1.0, 0.0))
                return y

            lax.fori_loop(0, PASS // 16, sub, 0)
            return x

        lax.fori_loop(0, n_pass, step, 0)
        pltpu.sync_copy(out_v, out_hbm.at[pl.ds(wid * b_w, b_w)])

    return pl.kernel(
        body,
        out_type=jax.ShapeDtypeStruct((B,), jnp.float32),
        mesh=mesh,
        compiler_params=pltpu.CompilerParams(
            use_tc_tiling_on_sc=False, needs_layout_passes=False),
        scratch_types=[
            pltpu.VMEM((3 * b_w,), jnp.int32),
            pltpu.VMEM((2, 3, PASS), jnp.int32),
            pltpu.VMEM((2, 3, PASS), jnp.int32),
            pltpu.VMEM((2, PASS, 128), jnp.float32),
            pltpu.VMEM((2, PASS, 128), jnp.float32),
            pltpu.VMEM((2, PASS, 128), jnp.float32),
            pltpu.VMEM((b_w,), jnp.float32),
            pltpu.SemaphoreType.DMA,
        ],
    )


def kernel(user_input, pos_item_input, neg_item_input, user_table, item_table):
    B = user_input.shape[0]
    V = user_table.shape[0]
    b_w = B // NW
    idx = jnp.concatenate(
        [user_input.astype(jnp.int32).reshape(NW, b_w),
         pos_item_input.astype(jnp.int32).reshape(NW, b_w),
         neg_item_input.astype(jnp.int32).reshape(NW, b_w)], axis=1)
    tab_u, tab_i = _repack_fn(V)(user_table.T, item_table.T)
    out = _fused_fn(B)(idx, tab_u, tab_i)
    return out.reshape(B, 1)


# speedup vs baseline: 3.0152x; 1.0284x over previous
import functools

import jax
import jax.numpy as jnp
from jax import lax
from jax.experimental import pallas as pl
from jax.experimental.pallas import tpu as pltpu
from jax.experimental.pallas import tpu_sc as plsc

DIM = 32
NC, NS = 2, 16
NW = NC * NS
SUBB = 16
TBLK = 4 * SUBB * 512
PASS = 128


def _rsqrt16(x):
    i = plsc.bitcast(x, jnp.int32)
    i = jnp.int32(0x5F3759DF) - (i >> 1)
    y = plsc.bitcast(i, jnp.float32)
    for _ in range(3):
        y = y * (1.5 - 0.5 * x * y * y)
    return y


def _repack_body(u_ref, i_ref, ou_ref, oi_ref):
    for ref, out in ((u_ref, ou_ref), (i_ref, oi_ref)):
        for i in range(TBLK // 512):
            y = jnp.concatenate(
                [ref[:, pl.ds((4 * i + s) * 128, 128)] for s in range(4)],
                axis=0)
            out[pl.ds(i * 128, 128), :] = y.T


@functools.cache
def _repack_fn(V):
    grid = (V + TBLK - 1) // TBLK
    return pl.pallas_call(
        _repack_body,
        grid=(grid,),
        in_specs=[pl.BlockSpec((DIM, TBLK), lambda i: (0, i))] * 2,
        out_specs=[pl.BlockSpec((SUBB * 512, 128), lambda i: (i, 0))] * 2,
        out_shape=[jax.ShapeDtypeStruct(
            (grid * SUBB * 512, 128), jnp.float32)] * 2,
    )


@functools.cache
def _fused_fn(B):
    b_w = B // NW
    n_pass = b_w // PASS
    mesh = plsc.VectorSubcoreMesh(
        core_axis_name="c", subcore_axis_name="s",
        num_cores=NC, num_subcores=NS)

    def body(idx_hbm, utab_hbm, itab_hbm, out_hbm,
             idx_v, poff_v, widx, rows_u, rows_p, rows_n, out_v, sem):
        wid = lax.axis_index("s") * NC + lax.axis_index("c")
        pltpu.sync_copy(idx_hbm.at[wid], idx_v)

        tabs = ((utab_hbm, rows_u, 0), (itab_hbm, rows_p, 1),
                (itab_hbm, rows_n, 2))

        def gen_and_fire(p, buf):
            def gen(s, x):
                for _, _, t in tabs:
                    ids = idx_v[pl.ds(t * b_w + p * PASS + s * 16, 16)]
                    row = ((ids >> 9) << 7) + (ids & 127)
                    poff = ((ids >> 7) & 3) << 5
                    widx[buf, t, pl.ds(s * 16, 16)] = row
                    poff_v[buf, t, pl.ds(s * 16, 16)] = poff
                return x
            lax.fori_loop(0, PASS // 16, gen, 0)
            for tab, rows, t in tabs:
                pltpu.async_copy(tab.at[widx.at[buf, t]], rows.at[buf], sem)

        gen_and_fire(0, 0)

        eps = jnp.float32(1e-12)
        lane = lax.iota(jnp.int32, 16)

        def step(p, x):
            buf = lax.rem(p, 2)
            nbuf = lax.rem(p + 1, 2)

            @pl.when(p + 1 < n_pass)
            def _():
                gen_and_fire(p + 1, nbuf)

            for tab, rows, t in tabs:
                pltpu.make_async_copy(
                    tab.at[widx.at[buf, t]], rows.at[buf], sem).wait()

            def sub(s, y):
                pu = poff_v[buf, 0, pl.ds(s * 16, 16)]
                pp_ = poff_v[buf, 1, pl.ds(s * 16, 16)]
                pn = poff_v[buf, 2, pl.ds(s * 16, 16)]
                zero = jnp.zeros((16,), jnp.float32)
                uu = pp = nn = up = un = zero
                for r in range(16):
                    il = s * 16 + r
                    u0 = rows_u[buf, il, pl.ds(pu[r], 16)]
                    u1 = rows_u[buf, il, pl.ds(pu[r] + 16, 16)]
                    p0 = rows_p[buf, il, pl.ds(pp_[r], 16)]
                    p1 = rows_p[buf, il, pl.ds(pp_[r] + 16, 16)]
                    n0 = rows_n[buf, il, pl.ds(pn[r], 16)]
                    n1 = rows_n[buf, il, pl.ds(pn[r] + 16, 16)]
                    m = lane == r
                    uu = jnp.where(m, jnp.sum(u0 * u0 + u1 * u1), uu)
                    pp = jnp.where(m, jnp.sum(p0 * p0 + p1 * p1), pp)
                    nn = jnp.where(m, jnp.sum(n0 * n0 + n1 * n1), nn)
                    up = jnp.where(m, jnp.sum(u0 * p0 + u1 * p1), up)
                    un = jnp.where(m, jnp.sum(u0 * n0 + u1 * n1), un)
                uu = jnp.maximum(uu, eps)
                pos = up * _rsqrt16(uu * jnp.maximum(pp, eps))
                neg = un * _rsqrt16(uu * jnp.maximum(nn, eps))
                out_v[pl.ds(p * PASS + s * 16, 16)] = (
                    jnp.maximum(neg - pos + 1.0, 0.0))
                return y

            lax.fori_loop(0, PASS // 16, sub, 0)
            return x

        lax.fori_loop(0, n_pass, step, 0)
        pltpu.sync_copy(out_v, out_hbm.at[pl.ds(wid * b_w, b_w)])

    return pl.kernel(
        body,
        out_type=jax.ShapeDtypeStruct((B,), jnp.float32),
        mesh=mesh,
        compiler_params=pltpu.CompilerParams(
            use_tc_tiling_on_sc=False, needs_layout_passes=False),
        scratch_types=[
            pltpu.VMEM((3 * b_w,), jnp.int32),
            pltpu.VMEM((2, 3, PASS), jnp.int32),
            pltpu.VMEM((2, 3, PASS), jnp.int32),
            pltpu.VMEM((2, PASS, 128), jnp.float32),
            pltpu.VMEM((2, PASS, 128), jnp.float32),
            pltpu.VMEM((2, PASS, 128), jnp.float32),
            pltpu.VMEM((b_w,), jnp.float32),
            pltpu.SemaphoreType.DMA,
        ],
    )


def kernel(user_input, pos_item_input, neg_item_input, user_table, item_table):
    B = user_input.shape[0]
    V = user_table.shape[0]
    b_w = B // NW
    idx = jnp.concatenate(
        [user_input.astype(jnp.int32).reshape(NW, b_w),
         pos_item_input.astype(jnp.int32).reshape(NW, b_w),
         neg_item_input.astype(jnp.int32).reshape(NW, b_w)], axis=1)
    tab_u, tab_i = _repack_fn(V)(user_table.T, item_table.T)
    out = _fused_fn(B)(idx, tab_u, tab_i)
    return out.reshape(B, 1)
